# baseline (speedup 1.0000x reference)
import functools

import jax
import jax.numpy as jnp
from jax.experimental import pallas as pl

_SOS_TOKEN = 1
_BEAM_WIDTH = 10
_MAX_STEPS = 6
_TV = 2048


def _gru_cell(x, h, wih, whh, b):
    gi = x @ wih + b
    gh = h @ whh
    ir, iz, inn = jnp.split(gi, 3, axis=-1)
    hr, hz, hn = jnp.split(gh, 3, axis=-1)
    r = jax.nn.sigmoid(ir + hr)
    z = jax.nn.sigmoid(iz + hz)
    n = jnp.tanh(inn + r * hn)
    return (1.0 - z) * n + z * h


def _logits_kernel(h_ref, wout_ref, bout_ref, logits_ref, segarg_ref, *,
                   vocab):
    i = pl.program_id(0)
    rk = h_ref.shape[0]
    nseg = _TV // 128
    logits = jnp.dot(h_ref[...], wout_ref[...],
                     preferred_element_type=jnp.float32) + bout_ref[...]
    col = i * _TV + jax.lax.broadcasted_iota(jnp.int32, (1, _TV), 1)
    logits = jnp.where(col < vocab, logits, -1e30)
    logits_ref[...] = logits
    l3 = logits.reshape(rk, nseg, 128)
    sm = jnp.max(l3, axis=2, keepdims=True)
    col3 = (i * _TV
            + jax.lax.broadcasted_iota(jnp.int32, (rk, nseg, 128), 1) * 128
            + jax.lax.broadcasted_iota(jnp.int32, (rk, nseg, 128), 2))
    segarg_ref[...] = jnp.min(
        jnp.where(l3 == sm, col3, jnp.int32(2**30)), axis=2).reshape(
            1, rk, nseg)


def kernel(input_seq, input_length, embed, enc_Wih, enc_Whh, enc_b,
           dec_Wih, dec_Whh, dec_b, W_out, b_out):
    b_sz, s_len = input_seq.shape
    d = embed.shape[1]
    vocab = W_out.shape[1]
    k = _BEAM_WIDTH
    rk = b_sz * k
    n_tiles = (vocab + _TV - 1) // _TV
    n_seg = n_tiles * (_TV // 128)

    x_emb = jnp.take(embed, input_seq, axis=0)

    def enc_step(h, inp):
        xt, t = inp
        h_new = _gru_cell(xt, h, enc_Wih, enc_Whh, enc_b)
        mask = (t < input_length)[:, None]
        return jnp.where(mask, h_new, h), None

    h0 = jnp.zeros((b_sz, d), dtype=embed.dtype)
    enc_h, _ = jax.lax.scan(
        enc_step, h0, (jnp.swapaxes(x_emb, 0, 1), jnp.arange(s_len)))

    h = jnp.broadcast_to(enc_h[:, None, :], (b_sz, k, d))
    scores = jnp.full((b_sz, k), -1e9, dtype=embed.dtype).at[:, 0].set(0.0)
    tokens = jnp.full((b_sz, k), _SOS_TOKEN, jnp.int32)

    logits_call = pl.pallas_call(
        functools.partial(_logits_kernel, vocab=vocab),
        grid=(n_tiles,),
        in_specs=[
            pl.BlockSpec((rk, d), lambda i: (0, 0)),
            pl.BlockSpec((d, _TV), lambda i: (0, i)),
            pl.BlockSpec((1, _TV), lambda i: (0, i)),
        ],
        out_specs=[
            pl.BlockSpec((rk, _TV), lambda i: (0, i)),
            pl.BlockSpec((1, rk, _TV // 128), lambda i: (i, 0, 0)),
        ],
        out_shape=[
            jax.ShapeDtypeStruct((rk, vocab), jnp.float32),
            jax.ShapeDtypeStruct((n_tiles, rk, _TV // 128), jnp.int32),
        ],
    )

    bout2 = b_out.reshape(1, -1)
    off128 = jax.lax.broadcasted_iota(jnp.int32, (1, k, 128), 2)
    toks_hist, parents_hist = [], []
    for _ in range(_MAX_STEPS):
        x = jnp.take(embed, tokens, axis=0)
        h = _gru_cell(x, h, dec_Wih, dec_Whh, dec_b)
        logits, segarg3 = logits_call(h.reshape(rk, d), W_out, bout2)
        segarg = segarg3.transpose(1, 0, 2).reshape(b_sz, k, n_seg)

        logits3 = logits.reshape(b_sz, k, vocab)
        candf = scores[:, :, None] + jax.nn.log_softmax(logits3, axis=-1)
        candf = jax.lax.optimization_barrier(candf)

        seg_cand = jnp.where(
            segarg < vocab,
            jnp.take_along_axis(candf, jnp.minimum(segarg, vocab - 1),
                                axis=2), -1e30)
        _, seg_idx = jax.lax.top_k(seg_cand.reshape(rk, n_seg), k)
        seg_idx = jnp.sort(seg_idx, axis=1)
        cols = (seg_idx.reshape(b_sz, k, k, 1) * 128
                + off128[:, :, None, :])
        colsf = cols.reshape(b_sz, k, k * 128)
        g_cand = jnp.where(
            colsf < vocab,
            jnp.take_along_axis(candf, jnp.minimum(colsf, vocab - 1),
                                axis=2), -1e30)
        tv_cand, tl = jax.lax.top_k(g_cand, k)
        ti = jnp.take_along_axis(colsf, tl, axis=2)

        cand = tv_cand.reshape(b_sz, k * k)
        scores, ci = jax.lax.top_k(cand, k)
        parent = ci // k
        tokens = jnp.take_along_axis(
            ti.reshape(b_sz, k * k), ci, axis=1).astype(jnp.int32)
        h = jnp.take_along_axis(h, parent[:, :, None], axis=1)
        toks_hist.append(tokens)
        parents_hist.append(parent)

    cur = jnp.broadcast_to(jnp.arange(k)[None, :], (b_sz, k))
    rev = []
    for t in range(_MAX_STEPS - 1, -1, -1):
        rev.append(jnp.take_along_axis(toks_hist[t], cur, axis=1))
        cur = jnp.take_along_axis(parents_hist[t], cur, axis=1)
    seqs = jnp.stack(rev[::-1], axis=-1)
    return seqs, scores
